# baseline (speedup 1.0000x reference)
import functools

import jax
import jax.numpy as jnp
from jax import lax
from jax.experimental import pallas as pl
from jax.experimental.pallas import tpu as pltpu
from jax.experimental.pallas import tpu_sc as plsc

N = 10000
D = 128
G = 64
E = 320000
NUM_LAYERS = 8

NC = 2
NS = 16
NW = NC * NS
EPW = E // NW
C = 120
NCHUNK = -(-EPW // C)
PAD = NCHUNK * C - EPW
NP = 10240
RPS = NP // NS
RCH = 80
NRCH = RPS // RCH

BLK = 1000
NBLK = N // BLK

_f32 = jnp.float32
_PREC = lax.Precision.HIGHEST


def _segsum_body(hs_hbm, idx_hbm, zrow_hbm, out_hbm,
                 i0, i1, i2, i3, b0, b1, b2, acc,
                 is0, is1, is2, is3, gs0, gs1, gs2, ss0, ss1, ss2):
    cid = lax.axis_index("c")
    sid = lax.axis_index("s")
    wid = sid * NC + cid
    ibufs = (i0, i1, i2, i3)
    isems = (is0, is1, is2, is3)
    bufs = (b0, b1, b2)
    gsems = (gs0, gs1, gs2)
    ssems = (ss0, ss1, ss2)

    def idx_load(k):
        return pltpu.async_copy(idx_hbm.at[wid, k], ibufs[k % 4],
                                isems[k % 4])

    def gather(k):
        return pltpu.async_copy(hs_hbm.at[ibufs[k % 4].at[0]], bufs[k % 3],
                                gsems[k % 3])

    def scatter(k):
        return pltpu.async_copy(bufs[k % 3], acc.at[ibufs[k % 4].at[1]],
                                ssems[k % 3], add=True)

    idx_load(0).wait()
    gather(0)
    idx_load(1)
    idx_load(2)
    base = sid * RPS
    pltpu.sync_copy(zrow_hbm, acc.at[pl.ds(base, RPS)])
    pltpu.make_async_copy(idx_hbm.at[wid, 1], i1, is1).wait()
    gather(1)
    plsc.subcore_barrier()
    for j in range(NCHUNK):
        g = j + 2
        if g < NCHUNK:
            pltpu.make_async_copy(idx_hbm.at[wid, g], ibufs[g % 4],
                                  isems[g % 4]).wait()
            if g >= 3:
                pltpu.make_async_copy(bufs[g % 3],
                                      acc.at[ibufs[(g - 3) % 4].at[1]],
                                      ssems[g % 3]).wait()
            gather(g)
        pltpu.make_async_copy(hs_hbm.at[ibufs[j % 4].at[0]], bufs[j % 3],
                              gsems[j % 3]).wait()
        scatter(j)
        if j + 3 < NCHUNK:
            idx_load(j + 3)
    for k in range(max(0, NCHUNK - 3), NCHUNK):
        pltpu.make_async_copy(bufs[k % 3], acc.at[ibufs[k % 4].at[1]],
                              ssems[k % 3]).wait()
    plsc.subcore_barrier()
    pltpu.sync_copy(acc.at[pl.ds(base, RPS)],
                    out_hbm.at[cid, pl.ds(base, RPS)])


_sc_segsum = pl.kernel(
    _segsum_body,
    out_type=jax.ShapeDtypeStruct((NC, NP, D), _f32),
    mesh=plsc.VectorSubcoreMesh(core_axis_name="c", subcore_axis_name="s",
                                num_cores=NC, num_subcores=NS),
    scratch_types=(
        [pltpu.VMEM((2, C), jnp.int32)] * 4
        + [pltpu.VMEM((C, D), _f32)] * 3
        + [pltpu.VMEM_SHARED((NP, D), _f32)]
        + [pltpu.SemaphoreType.DMA] * 10
    ),
)


def _dinv_body(deg_ref, o_ref):
    o_ref[...] = lax.rsqrt(deg_ref[0] + deg_ref[1] + 1.0)


_pc_dinv = pl.pallas_call(
    _dinv_body,
    grid=(NBLK,),
    in_specs=[pl.BlockSpec((NC, BLK, D), lambda i: (0, i, 0))],
    out_specs=pl.BlockSpec((BLK, D), lambda i: (i, 0)),
    out_shape=jax.ShapeDtypeStruct((N, D), _f32),
)


def _init_body(x_ref, w0_ref, b0_ref, w1_ref, dinv_ref, o_ref):
    h0 = jnp.dot(x_ref[...], w0_ref[...], precision=_PREC,
                 preferred_element_type=_f32) + b0_ref[0:1, :]
    o_ref[...] = jnp.dot(h0, w1_ref[...], precision=_PREC,
                         preferred_element_type=_f32) * dinv_ref[...]


_pc_init = pl.pallas_call(
    _init_body,
    grid=(NBLK,),
    in_specs=[
        pl.BlockSpec((BLK, D), lambda i: (i, 0)),
        pl.BlockSpec((D, D), lambda i: (0, 0)),
        pl.BlockSpec((8, D), lambda i: (0, 0)),
        pl.BlockSpec((D, D), lambda i: (0, 0)),
        pl.BlockSpec((BLK, D), lambda i: (i, 0)),
    ],
    out_specs=pl.BlockSpec((BLK, D), lambda i: (i, 0)),
    out_shape=jax.ShapeDtypeStruct((N, D), _f32),
)


def _post_agg(p_ref, hs_ref, dinv_ref, pars_ref):
    agg = dinv_ref[...] * (p_ref[0] + p_ref[1] + hs_ref[...]) + pars_ref[0:1, :]
    mu = jnp.mean(agg, axis=-1, keepdims=True)
    xc = agg - mu
    var = jnp.mean(xc * xc, axis=-1, keepdims=True)
    hn = xc * lax.rsqrt(var + 1e-5) * pars_ref[1:2, :] + pars_ref[2:3, :]
    return jnp.maximum(hn, 0.0)


def _layer_body(p_ref, hs_ref, dinv_ref, w_ref, pars_ref, o_ref):
    h = _post_agg(p_ref, hs_ref, dinv_ref, pars_ref)
    o_ref[...] = jnp.dot(h, w_ref[...], precision=_PREC,
                         preferred_element_type=_f32) * dinv_ref[...]


_pc_layer = pl.pallas_call(
    _layer_body,
    grid=(NBLK,),
    in_specs=[
        pl.BlockSpec((NC, BLK, D), lambda i: (0, i, 0)),
        pl.BlockSpec((BLK, D), lambda i: (i, 0)),
        pl.BlockSpec((BLK, D), lambda i: (i, 0)),
        pl.BlockSpec((D, D), lambda i: (0, 0)),
        pl.BlockSpec((8, D), lambda i: (0, 0)),
    ],
    out_specs=pl.BlockSpec((BLK, D), lambda i: (i, 0)),
    out_shape=jax.ShapeDtypeStruct((N, D), _f32),
)


def _final_body(p_ref, hs_ref, dinv_ref, pars_ref, batch_ref, wf_ref, bf_ref,
                o_ref, pool_ref, cnt_ref):
    i = pl.program_id(0)

    @pl.when(i == 0)
    def _():
        pool_ref[...] = jnp.zeros_like(pool_ref)
        cnt_ref[...] = jnp.zeros_like(cnt_ref)

    h = _post_agg(p_ref, hs_ref, dinv_ref, pars_ref)
    ids = batch_ref[:, 0:1]
    gi = lax.broadcasted_iota(jnp.int32, (BLK, G), 1)
    oh = (ids == gi).astype(_f32)
    dn = (((0,), (0,)), ((), ()))
    pool_ref[...] += lax.dot_general(oh, h, dn, precision=_PREC,
                                     preferred_element_type=_f32)
    cnt_ref[...] += lax.dot_general(oh, jnp.ones((BLK, D), _f32), dn,
                                    precision=_PREC,
                                    preferred_element_type=_f32)

    @pl.when(i == NBLK - 1)
    def _():
        pooled = pool_ref[...] / jnp.maximum(cnt_ref[...], 1.0)
        o_ref[...] = jnp.dot(pooled, wf_ref[...], precision=_PREC,
                             preferred_element_type=_f32) + bf_ref[0:1, :]


_pc_final = pl.pallas_call(
    _final_body,
    grid=(NBLK,),
    in_specs=[
        pl.BlockSpec((NC, BLK, D), lambda i: (0, i, 0)),
        pl.BlockSpec((BLK, D), lambda i: (i, 0)),
        pl.BlockSpec((BLK, D), lambda i: (i, 0)),
        pl.BlockSpec((8, D), lambda i: (0, 0)),
        pl.BlockSpec((BLK, D), lambda i: (i, 0)),
        pl.BlockSpec((D, D), lambda i: (0, 0)),
        pl.BlockSpec((8, D), lambda i: (0, 0)),
    ],
    out_specs=pl.BlockSpec((G, D), lambda i: (0, 0)),
    out_shape=jax.ShapeDtypeStruct((G, D), _f32),
    scratch_shapes=[pltpu.VMEM((G, D), _f32), pltpu.VMEM((G, D), _f32)],
)


def _row8(v):
    return jnp.broadcast_to(v[None, :], (8, D)).astype(_f32)


def _pack3(b, g, be):
    return jnp.concatenate(
        [b[None, :], g[None, :], be[None, :], jnp.zeros((5, D), _f32)], axis=0)


def kernel(x, edge_index, batch, W0, b0, Ws, bs, gammas, betas, Wf, bf):
    srcw = edge_index[0].reshape(NW, EPW)
    dstw = edge_index[1].reshape(NW, EPW)
    wi = jnp.arange(NW, dtype=jnp.int32)[:, None]
    pi = jnp.arange(PAD, dtype=jnp.int32)[None, :]
    src_pad = (wi * 131 + pi * 97) % N
    dst_pad = N + (wi * 7 + pi * 13) % (NP - N)
    srcp = jnp.concatenate([srcw, src_pad], axis=1).reshape(NW, NCHUNK, C)
    dstp = jnp.concatenate([dstw, dst_pad], axis=1).reshape(NW, NCHUNK, C)
    idxc = jnp.stack([srcp, dstp], axis=2)
    zrow = jnp.zeros((RPS, D), _f32)
    batchb = jnp.broadcast_to(batch[:, None], (N, D))

    deg2 = _sc_segsum(jnp.ones((N, D), _f32), idxc, zrow)
    dinv = _pc_dinv(deg2)
    hs = _pc_init(x, W0, _row8(b0), Ws[0], dinv)
    for i in range(NUM_LAYERS):
        p2 = _sc_segsum(hs, idxc, zrow)
        if i + 1 < NUM_LAYERS:
            hs = _pc_layer(p2, hs, dinv, Ws[i + 1],
                           _pack3(bs[i], gammas[i], betas[i]))
        else:
            out = _pc_final(p2, hs, dinv, _pack3(bs[i], gammas[i], betas[i]),
                            batchb, Wf, _row8(bf))
    return out


# speedup vs baseline: 1.0006x; 1.0006x over previous
import functools

import jax
import jax.numpy as jnp
from jax import lax
from jax.experimental import pallas as pl
from jax.experimental.pallas import tpu as pltpu
from jax.experimental.pallas import tpu_sc as plsc

N = 10000
D = 128
G = 64
E = 320000
NUM_LAYERS = 8

NC = 2
NS = 16
NW = NC * NS
EPW = E // NW
C = 120
NCHUNK = -(-EPW // C)
PAD = NCHUNK * C - EPW
NP = 10240
RPS = NP // NS
RCH = 80
NRCH = RPS // RCH

BLK = 1000
NBLK = N // BLK

_f32 = jnp.float32
_PREC = lax.Precision.HIGHEST


def _segsum_body(hs_hbm, idx_hbm, zrow_hbm, out_hbm,
                 i0, i1, i2, i3, b0, b1, b2, acc,
                 is0, is1, is2, is3, gs0, gs1, gs2, ss0, ss1, ss2):
    cid = lax.axis_index("c")
    sid = lax.axis_index("s")
    wid = sid * NC + cid
    ibufs = (i0, i1, i2, i3)
    isems = (is0, is1, is2, is3)
    bufs = (b0, b1, b2)
    gsems = (gs0, gs1, gs2)
    ssems = (ss0, ss1, ss2)

    def idx_load(k):
        return pltpu.async_copy(idx_hbm.at[wid, k], ibufs[k % 4],
                                isems[k % 4])

    def gather(k):
        return pltpu.async_copy(hs_hbm.at[ibufs[k % 4].at[0]], bufs[k % 3],
                                gsems[k % 3])

    def scatter(k):
        return pltpu.async_copy(bufs[k % 3], acc.at[ibufs[k % 4].at[1]],
                                ssems[k % 3], add=True)

    idx_load(0).wait()
    gather(0)
    idx_load(1)
    idx_load(2)
    base = sid * RPS
    stg = b2.at[pl.ds(0, RCH)]
    pltpu.sync_copy(zrow_hbm, stg)
    for r in range(NRCH):
        pltpu.sync_copy(stg, acc.at[pl.ds(base + r * RCH, RCH)])
    pltpu.make_async_copy(idx_hbm.at[wid, 1], i1, is1).wait()
    gather(1)
    plsc.subcore_barrier()
    for j in range(NCHUNK):
        g = j + 2
        if g < NCHUNK:
            pltpu.make_async_copy(idx_hbm.at[wid, g], ibufs[g % 4],
                                  isems[g % 4]).wait()
            if g >= 3:
                pltpu.make_async_copy(bufs[g % 3],
                                      acc.at[ibufs[(g - 3) % 4].at[1]],
                                      ssems[g % 3]).wait()
            gather(g)
        pltpu.make_async_copy(hs_hbm.at[ibufs[j % 4].at[0]], bufs[j % 3],
                              gsems[j % 3]).wait()
        scatter(j)
        if j + 3 < NCHUNK:
            idx_load(j + 3)
    for k in range(max(0, NCHUNK - 3), NCHUNK):
        pltpu.make_async_copy(bufs[k % 3], acc.at[ibufs[k % 4].at[1]],
                              ssems[k % 3]).wait()
    plsc.subcore_barrier()
    for r in range(NRCH):
        pltpu.sync_copy(acc.at[pl.ds(base + r * RCH, RCH)], stg)
        pltpu.sync_copy(stg, out_hbm.at[cid, pl.ds(base + r * RCH, RCH)])


_sc_segsum = pl.kernel(
    _segsum_body,
    out_type=jax.ShapeDtypeStruct((NC, NP, D), _f32),
    mesh=plsc.VectorSubcoreMesh(core_axis_name="c", subcore_axis_name="s",
                                num_cores=NC, num_subcores=NS),
    scratch_types=(
        [pltpu.VMEM((2, C), jnp.int32)] * 4
        + [pltpu.VMEM((C, D), _f32)] * 3
        + [pltpu.VMEM_SHARED((NP, D), _f32)]
        + [pltpu.SemaphoreType.DMA] * 10
    ),
)


def _dinv_body(deg_ref, o_ref):
    o_ref[...] = lax.rsqrt(deg_ref[0] + deg_ref[1] + 1.0)


_pc_dinv = pl.pallas_call(
    _dinv_body,
    grid=(NBLK,),
    in_specs=[pl.BlockSpec((NC, BLK, D), lambda i: (0, i, 0))],
    out_specs=pl.BlockSpec((BLK, D), lambda i: (i, 0)),
    out_shape=jax.ShapeDtypeStruct((N, D), _f32),
)


def _init_body(x_ref, w0_ref, b0_ref, w1_ref, dinv_ref, o_ref):
    h0 = jnp.dot(x_ref[...], w0_ref[...], precision=_PREC,
                 preferred_element_type=_f32) + b0_ref[0:1, :]
    o_ref[...] = jnp.dot(h0, w1_ref[...], precision=_PREC,
                         preferred_element_type=_f32) * dinv_ref[...]


_pc_init = pl.pallas_call(
    _init_body,
    grid=(NBLK,),
    in_specs=[
        pl.BlockSpec((BLK, D), lambda i: (i, 0)),
        pl.BlockSpec((D, D), lambda i: (0, 0)),
        pl.BlockSpec((8, D), lambda i: (0, 0)),
        pl.BlockSpec((D, D), lambda i: (0, 0)),
        pl.BlockSpec((BLK, D), lambda i: (i, 0)),
    ],
    out_specs=pl.BlockSpec((BLK, D), lambda i: (i, 0)),
    out_shape=jax.ShapeDtypeStruct((N, D), _f32),
)


def _post_agg(p_ref, hs_ref, dinv_ref, pars_ref):
    agg = dinv_ref[...] * (p_ref[0] + p_ref[1] + hs_ref[...]) + pars_ref[0:1, :]
    mu = jnp.mean(agg, axis=-1, keepdims=True)
    xc = agg - mu
    var = jnp.mean(xc * xc, axis=-1, keepdims=True)
    hn = xc * lax.rsqrt(var + 1e-5) * pars_ref[1:2, :] + pars_ref[2:3, :]
    return jnp.maximum(hn, 0.0)


def _layer_body(p_ref, hs_ref, dinv_ref, w_ref, pars_ref, o_ref):
    h = _post_agg(p_ref, hs_ref, dinv_ref, pars_ref)
    o_ref[...] = jnp.dot(h, w_ref[...], precision=_PREC,
                         preferred_element_type=_f32) * dinv_ref[...]


_pc_layer = pl.pallas_call(
    _layer_body,
    grid=(NBLK,),
    in_specs=[
        pl.BlockSpec((NC, BLK, D), lambda i: (0, i, 0)),
        pl.BlockSpec((BLK, D), lambda i: (i, 0)),
        pl.BlockSpec((BLK, D), lambda i: (i, 0)),
        pl.BlockSpec((D, D), lambda i: (0, 0)),
        pl.BlockSpec((8, D), lambda i: (0, 0)),
    ],
    out_specs=pl.BlockSpec((BLK, D), lambda i: (i, 0)),
    out_shape=jax.ShapeDtypeStruct((N, D), _f32),
)


def _final_body(p_ref, hs_ref, dinv_ref, pars_ref, batch_ref, wf_ref, bf_ref,
                o_ref, pool_ref, cnt_ref):
    i = pl.program_id(0)

    @pl.when(i == 0)
    def _():
        pool_ref[...] = jnp.zeros_like(pool_ref)
        cnt_ref[...] = jnp.zeros_like(cnt_ref)

    h = _post_agg(p_ref, hs_ref, dinv_ref, pars_ref)
    ids = batch_ref[:, 0:1]
    gi = lax.broadcasted_iota(jnp.int32, (BLK, G), 1)
    oh = (ids == gi).astype(_f32)
    dn = (((0,), (0,)), ((), ()))
    pool_ref[...] += lax.dot_general(oh, h, dn, precision=_PREC,
                                     preferred_element_type=_f32)
    cnt_ref[...] += lax.dot_general(oh, jnp.ones((BLK, D), _f32), dn,
                                    precision=_PREC,
                                    preferred_element_type=_f32)

    @pl.when(i == NBLK - 1)
    def _():
        pooled = pool_ref[...] / jnp.maximum(cnt_ref[...], 1.0)
        o_ref[...] = jnp.dot(pooled, wf_ref[...], precision=_PREC,
                             preferred_element_type=_f32) + bf_ref[0:1, :]


_pc_final = pl.pallas_call(
    _final_body,
    grid=(NBLK,),
    in_specs=[
        pl.BlockSpec((NC, BLK, D), lambda i: (0, i, 0)),
        pl.BlockSpec((BLK, D), lambda i: (i, 0)),
        pl.BlockSpec((BLK, D), lambda i: (i, 0)),
        pl.BlockSpec((8, D), lambda i: (0, 0)),
        pl.BlockSpec((BLK, D), lambda i: (i, 0)),
        pl.BlockSpec((D, D), lambda i: (0, 0)),
        pl.BlockSpec((8, D), lambda i: (0, 0)),
    ],
    out_specs=pl.BlockSpec((G, D), lambda i: (0, 0)),
    out_shape=jax.ShapeDtypeStruct((G, D), _f32),
    scratch_shapes=[pltpu.VMEM((G, D), _f32), pltpu.VMEM((G, D), _f32)],
)


def _row8(v):
    return jnp.broadcast_to(v[None, :], (8, D)).astype(_f32)


def _pack3(b, g, be):
    return jnp.concatenate(
        [b[None, :], g[None, :], be[None, :], jnp.zeros((5, D), _f32)], axis=0)


def kernel(x, edge_index, batch, W0, b0, Ws, bs, gammas, betas, Wf, bf):
    srcw = edge_index[0].reshape(NW, EPW)
    dstw = edge_index[1].reshape(NW, EPW)
    wi = jnp.arange(NW, dtype=jnp.int32)[:, None]
    pi = jnp.arange(PAD, dtype=jnp.int32)[None, :]
    src_pad = (wi * 131 + pi * 97) % N
    dst_pad = N + (wi * 7 + pi * 13) % (NP - N)
    srcp = jnp.concatenate([srcw, src_pad], axis=1).reshape(NW, NCHUNK, C)
    dstp = jnp.concatenate([dstw, dst_pad], axis=1).reshape(NW, NCHUNK, C)
    idxc = jnp.stack([srcp, dstp], axis=2)
    zrow = jnp.zeros((RCH, D), _f32)
    batchb = jnp.broadcast_to(batch[:, None], (N, D))

    deg2 = _sc_segsum(jnp.ones((N, D), _f32), idxc, zrow)
    dinv = _pc_dinv(deg2)
    hs = _pc_init(x, W0, _row8(b0), Ws[0], dinv)
    for i in range(NUM_LAYERS):
        p2 = _sc_segsum(hs, idxc, zrow)
        if i + 1 < NUM_LAYERS:
            hs = _pc_layer(p2, hs, dinv, Ws[i + 1],
                           _pack3(bs[i], gammas[i], betas[i]))
        else:
            out = _pc_final(p2, hs, dinv, _pack3(bs[i], gammas[i], betas[i]),
                            batchb, Wf, _row8(bf))
    return out


# speedup vs baseline: 1.0241x; 1.0235x over previous
import functools

import jax
import jax.numpy as jnp
from jax import lax
from jax.experimental import pallas as pl
from jax.experimental.pallas import tpu as pltpu
from jax.experimental.pallas import tpu_sc as plsc

N = 10000
D = 128
G = 64
E = 320000
NUM_LAYERS = 8

NC = 2
NS = 16
NW = NC * NS
EPW = E // NW
C = 120
NCHUNK = -(-EPW // C)
PAD = NCHUNK * C - EPW
NP = 10240
RPS = NP // NS
RCH = 80
NRCH = RPS // RCH

BLK = 1000
NBLK = N // BLK

_f32 = jnp.float32
_PREC = lax.Precision.HIGHEST


def _segsum_body(hs_hbm, idx_hbm, zrow_hbm, out_hbm,
                 i0, i1, i2, i3, b0, b1, b2, acc,
                 is0, is1, is2, is3, gs0, gs1, gs2, ss0, ss1, ss2):
    cid = lax.axis_index("c")
    sid = lax.axis_index("s")
    wid = sid * NC + cid
    ibufs = (i0, i1, i2, i3)
    isems = (is0, is1, is2, is3)
    bufs = (b0, b1, b2)
    gsems = (gs0, gs1, gs2)
    ssems = (ss0, ss1, ss2)

    def idx_load(k):
        return pltpu.async_copy(idx_hbm.at[wid, k], ibufs[k % 4],
                                isems[k % 4])

    def gather(k):
        return pltpu.async_copy(hs_hbm.at[ibufs[k % 4].at[0]], bufs[k % 3],
                                gsems[k % 3])

    def scatter(k):
        return pltpu.async_copy(bufs[k % 3], acc.at[ibufs[k % 4].at[1]],
                                ssems[k % 3], add=True)

    idx_load(0).wait()
    gather(0)
    idx_load(1)
    idx_load(2)
    base = sid * RPS
    stg = b2.at[pl.ds(0, RCH)]
    pltpu.sync_copy(zrow_hbm, stg)
    for r in range(NRCH):
        pltpu.sync_copy(stg, acc.at[pl.ds(base + r * RCH, RCH)])
    pltpu.make_async_copy(idx_hbm.at[wid, 1], i1, is1).wait()
    gather(1)
    plsc.subcore_barrier()
    for j in range(NCHUNK):
        g = j + 2
        if g < NCHUNK:
            pltpu.make_async_copy(idx_hbm.at[wid, g], ibufs[g % 4],
                                  isems[g % 4]).wait()
            if g >= 3:
                pltpu.make_async_copy(bufs[g % 3],
                                      acc.at[ibufs[(g - 3) % 4].at[1]],
                                      ssems[g % 3]).wait()
            gather(g)
        pltpu.make_async_copy(hs_hbm.at[ibufs[j % 4].at[0]], bufs[j % 3],
                              gsems[j % 3]).wait()
        scatter(j)
        if j + 3 < NCHUNK:
            idx_load(j + 3)
    for k in range(max(0, NCHUNK - 3), NCHUNK):
        pltpu.make_async_copy(bufs[k % 3], acc.at[ibufs[k % 4].at[1]],
                              ssems[k % 3]).wait()
    plsc.subcore_barrier()
    for r in range(NRCH):
        pltpu.sync_copy(acc.at[pl.ds(base + r * RCH, RCH)], stg)
        pltpu.sync_copy(stg, out_hbm.at[cid, pl.ds(base + r * RCH, RCH)])


_sc_segsum = pl.kernel(
    _segsum_body,
    out_type=jax.ShapeDtypeStruct((NC, NP, D), _f32),
    mesh=plsc.VectorSubcoreMesh(core_axis_name="c", subcore_axis_name="s",
                                num_cores=NC, num_subcores=NS),
    scratch_types=(
        [pltpu.VMEM((2, C), jnp.int32)] * 4
        + [pltpu.VMEM((C, D), _f32)] * 3
        + [pltpu.VMEM_SHARED((NP, D), _f32)]
        + [pltpu.SemaphoreType.DMA] * 10
    ),
)


def _count_body(idx_hbm, ones_hbm, zrow_hbm, out_hbm,
                i0, i1, i2, i3, ob, accd,
                is0, is1, is2, is3, ss0, ss1, ss2, ss3):
    cid = lax.axis_index("c")
    sid = lax.axis_index("s")
    wid = sid * NC + cid
    ibufs = (i0, i1, i2, i3)
    isems = (is0, is1, is2, is3)
    ssems = (ss0, ss1, ss2, ss3)

    def idx_load(k):
        return pltpu.async_copy(idx_hbm.at[wid, k], ibufs[k % 4],
                                isems[k % 4])

    def scat_desc(k):
        return pltpu.make_async_copy(ob, accd.at[ibufs[k % 4].at[1]],
                                     ssems[k % 4])

    idx_load(0)
    idx_load(1)
    idx_load(2)
    stg = ob.at[pl.ds(0, RCH)]
    pltpu.sync_copy(zrow_hbm, stg)
    base = sid * RPS
    for r in range(NRCH):
        pltpu.sync_copy(stg, accd.at[pl.ds(base + r * RCH, RCH)])
    pltpu.sync_copy(ones_hbm, ob)
    plsc.subcore_barrier()
    for j in range(NCHUNK):
        pltpu.make_async_copy(idx_hbm.at[wid, j], ibufs[j % 4],
                              isems[j % 4]).wait()
        pltpu.async_copy(ob, accd.at[ibufs[j % 4].at[1]], ssems[j % 4],
                         add=True)
        if j + 3 < NCHUNK:
            if j >= 1:
                scat_desc(j - 1).wait()
            idx_load(j + 3)
    for k in range(max(0, NCHUNK - 4), NCHUNK):
        scat_desc(k).wait()
    plsc.subcore_barrier()
    for r in range(NRCH):
        pltpu.sync_copy(accd.at[pl.ds(base + r * RCH, RCH)], stg)
        pltpu.sync_copy(stg, out_hbm.at[cid, pl.ds(base + r * RCH, RCH)])


_sc_count = pl.kernel(
    _count_body,
    out_type=jax.ShapeDtypeStruct((NC, NP, D), _f32),
    mesh=plsc.VectorSubcoreMesh(core_axis_name="c", subcore_axis_name="s",
                                num_cores=NC, num_subcores=NS),
    scratch_types=(
        [pltpu.VMEM((2, C), jnp.int32)] * 4
        + [pltpu.VMEM((C, D), _f32)]
        + [pltpu.VMEM_SHARED((NP, D), _f32)]
        + [pltpu.SemaphoreType.DMA] * 8
    ),
)


def _dinv_body(deg_ref, o_ref):
    o_ref[...] = lax.rsqrt(deg_ref[0] + deg_ref[1] + 1.0)


_pc_dinv = pl.pallas_call(
    _dinv_body,
    grid=(NBLK,),
    in_specs=[pl.BlockSpec((NC, BLK, D), lambda i: (0, i, 0))],
    out_specs=pl.BlockSpec((BLK, D), lambda i: (i, 0)),
    out_shape=jax.ShapeDtypeStruct((N, D), _f32),
)


def _init_body(x_ref, w0_ref, b0_ref, w1_ref, dinv_ref, o_ref):
    h0 = jnp.dot(x_ref[...], w0_ref[...], precision=_PREC,
                 preferred_element_type=_f32) + b0_ref[0:1, :]
    o_ref[...] = jnp.dot(h0, w1_ref[...], precision=_PREC,
                         preferred_element_type=_f32) * dinv_ref[...]


_pc_init = pl.pallas_call(
    _init_body,
    grid=(NBLK,),
    in_specs=[
        pl.BlockSpec((BLK, D), lambda i: (i, 0)),
        pl.BlockSpec((D, D), lambda i: (0, 0)),
        pl.BlockSpec((8, D), lambda i: (0, 0)),
        pl.BlockSpec((D, D), lambda i: (0, 0)),
        pl.BlockSpec((BLK, D), lambda i: (i, 0)),
    ],
    out_specs=pl.BlockSpec((BLK, D), lambda i: (i, 0)),
    out_shape=jax.ShapeDtypeStruct((N, D), _f32),
)


def _post_agg(p_ref, hs_ref, dinv_ref, pars_ref):
    agg = dinv_ref[...] * (p_ref[0] + p_ref[1] + hs_ref[...]) + pars_ref[0:1, :]
    mu = jnp.mean(agg, axis=-1, keepdims=True)
    xc = agg - mu
    var = jnp.mean(xc * xc, axis=-1, keepdims=True)
    hn = xc * lax.rsqrt(var + 1e-5) * pars_ref[1:2, :] + pars_ref[2:3, :]
    return jnp.maximum(hn, 0.0)


def _layer_body(p_ref, hs_ref, dinv_ref, w_ref, pars_ref, o_ref):
    h = _post_agg(p_ref, hs_ref, dinv_ref, pars_ref)
    o_ref[...] = jnp.dot(h, w_ref[...], precision=_PREC,
                         preferred_element_type=_f32) * dinv_ref[...]


_pc_layer = pl.pallas_call(
    _layer_body,
    grid=(NBLK,),
    in_specs=[
        pl.BlockSpec((NC, BLK, D), lambda i: (0, i, 0)),
        pl.BlockSpec((BLK, D), lambda i: (i, 0)),
        pl.BlockSpec((BLK, D), lambda i: (i, 0)),
        pl.BlockSpec((D, D), lambda i: (0, 0)),
        pl.BlockSpec((8, D), lambda i: (0, 0)),
    ],
    out_specs=pl.BlockSpec((BLK, D), lambda i: (i, 0)),
    out_shape=jax.ShapeDtypeStruct((N, D), _f32),
)


def _final_body(p_ref, hs_ref, dinv_ref, pars_ref, batch_ref, wf_ref, bf_ref,
                o_ref, pool_ref, cnt_ref):
    i = pl.program_id(0)

    @pl.when(i == 0)
    def _():
        pool_ref[...] = jnp.zeros_like(pool_ref)
        cnt_ref[...] = jnp.zeros_like(cnt_ref)

    h = _post_agg(p_ref, hs_ref, dinv_ref, pars_ref)
    ids = batch_ref[:, 0:1]
    gi = lax.broadcasted_iota(jnp.int32, (BLK, G), 1)
    oh = (ids == gi).astype(_f32)
    dn = (((0,), (0,)), ((), ()))
    pool_ref[...] += lax.dot_general(oh, h, dn, precision=_PREC,
                                     preferred_element_type=_f32)
    cnt_ref[...] += lax.dot_general(oh, jnp.ones((BLK, D), _f32), dn,
                                    precision=_PREC,
                                    preferred_element_type=_f32)

    @pl.when(i == NBLK - 1)
    def _():
        pooled = pool_ref[...] / jnp.maximum(cnt_ref[...], 1.0)
        o_ref[...] = jnp.dot(pooled, wf_ref[...], precision=_PREC,
                             preferred_element_type=_f32) + bf_ref[0:1, :]


_pc_final = pl.pallas_call(
    _final_body,
    grid=(NBLK,),
    in_specs=[
        pl.BlockSpec((NC, BLK, D), lambda i: (0, i, 0)),
        pl.BlockSpec((BLK, D), lambda i: (i, 0)),
        pl.BlockSpec((BLK, D), lambda i: (i, 0)),
        pl.BlockSpec((8, D), lambda i: (0, 0)),
        pl.BlockSpec((BLK, D), lambda i: (i, 0)),
        pl.BlockSpec((D, D), lambda i: (0, 0)),
        pl.BlockSpec((8, D), lambda i: (0, 0)),
    ],
    out_specs=pl.BlockSpec((G, D), lambda i: (0, 0)),
    out_shape=jax.ShapeDtypeStruct((G, D), _f32),
    scratch_shapes=[pltpu.VMEM((G, D), _f32), pltpu.VMEM((G, D), _f32)],
)


def _row8(v):
    return jnp.broadcast_to(v[None, :], (8, D)).astype(_f32)


def _pack3(b, g, be):
    return jnp.concatenate(
        [b[None, :], g[None, :], be[None, :], jnp.zeros((5, D), _f32)], axis=0)


def kernel(x, edge_index, batch, W0, b0, Ws, bs, gammas, betas, Wf, bf):
    srcw = edge_index[0].reshape(NW, EPW)
    dstw = edge_index[1].reshape(NW, EPW)
    wi = jnp.arange(NW, dtype=jnp.int32)[:, None]
    pi = jnp.arange(PAD, dtype=jnp.int32)[None, :]
    src_pad = (wi * 131 + pi * 97) % N
    dst_pad = N + (wi * 7 + pi * 13) % (NP - N)
    srcp = jnp.concatenate([srcw, src_pad], axis=1).reshape(NW, NCHUNK, C)
    dstp = jnp.concatenate([dstw, dst_pad], axis=1).reshape(NW, NCHUNK, C)
    idxc = jnp.stack([srcp, dstp], axis=2)
    zrow = jnp.zeros((RCH, D), _f32)
    batchb = jnp.broadcast_to(batch[:, None], (N, D))

    deg2 = _sc_count(idxc, jnp.ones((C, D), _f32), zrow)
    dinv = _pc_dinv(deg2)
    hs = _pc_init(x, W0, _row8(b0), Ws[0], dinv)
    for i in range(NUM_LAYERS):
        p2 = _sc_segsum(hs, idxc, zrow)
        if i + 1 < NUM_LAYERS:
            hs = _pc_layer(p2, hs, dinv, Ws[i + 1],
                           _pack3(bs[i], gammas[i], betas[i]))
        else:
            out = _pc_final(p2, hs, dinv, _pack3(bs[i], gammas[i], betas[i]),
                            batchb, Wf, _row8(bf))
    return out


# speedup vs baseline: 1.0539x; 1.0291x over previous
import functools

import jax
import jax.numpy as jnp
from jax import lax
from jax.experimental import pallas as pl
from jax.experimental.pallas import tpu as pltpu
from jax.experimental.pallas import tpu_sc as plsc

N = 10000
D = 128
G = 64
E = 320000
NUM_LAYERS = 8

NC = 2
NS = 16
NW = NC * NS
EPW = E // NW
C = 120
NCHUNK = -(-EPW // C)
PAD = NCHUNK * C - EPW
NP = 10240
RPS = NP // NS
RCH = 80
NRCH = RPS // RCH

BLK = 1000
NBLK = N // BLK

_f32 = jnp.float32
_PREC = lax.Precision.HIGHEST


def _segsum_body(hs_hbm, idx_hbm, zrow_hbm, out_hbm,
                 i0, i1, i2, i3, b0, b1, b2, acc,
                 is0, is1, is2, is3, gs0, gs1, gs2, ss0, ss1, ss2):
    cid = lax.axis_index("c")
    sid = lax.axis_index("s")
    wid = sid * NC + cid
    ibufs = (i0, i1, i2, i3)
    isems = (is0, is1, is2, is3)
    bufs = (b0, b1, b2)
    gsems = (gs0, gs1, gs2)
    ssems = (ss0, ss1, ss2)

    def idx_load(k):
        return pltpu.async_copy(idx_hbm.at[wid, k], ibufs[k % 4],
                                isems[k % 4])

    def gather(k):
        return pltpu.async_copy(hs_hbm.at[ibufs[k % 4].at[0]], bufs[k % 3],
                                gsems[k % 3])

    def scatter(k):
        return pltpu.async_copy(bufs[k % 3], acc.at[ibufs[k % 4].at[1]],
                                ssems[k % 3], add=True)

    idx_load(0).wait()
    gather(0)
    idx_load(1)
    idx_load(2)
    base = sid * RPS
    stg = b2.at[pl.ds(0, RCH)]
    pltpu.sync_copy(zrow_hbm, stg)
    for r in range(NRCH):
        pltpu.sync_copy(stg, acc.at[pl.ds(base + r * RCH, RCH)])
    pltpu.make_async_copy(idx_hbm.at[wid, 1], i1, is1).wait()
    gather(1)
    plsc.subcore_barrier()
    for j in range(NCHUNK):
        g = j + 2
        if g < NCHUNK:
            pltpu.make_async_copy(idx_hbm.at[wid, g], ibufs[g % 4],
                                  isems[g % 4]).wait()
            if g >= 3:
                pltpu.make_async_copy(bufs[g % 3],
                                      acc.at[ibufs[(g - 3) % 4].at[1]],
                                      ssems[g % 3]).wait()
            gather(g)
        pltpu.make_async_copy(hs_hbm.at[ibufs[j % 4].at[0]], bufs[j % 3],
                              gsems[j % 3]).wait()
        scatter(j)
        if j + 3 < NCHUNK:
            idx_load(j + 3)
    for k in range(max(0, NCHUNK - 3), NCHUNK):
        pltpu.make_async_copy(bufs[k % 3], acc.at[ibufs[k % 4].at[1]],
                              ssems[k % 3]).wait()
    plsc.subcore_barrier()
    for r in range(NRCH):
        pltpu.sync_copy(acc.at[pl.ds(base + r * RCH, RCH)], stg)
        pltpu.sync_copy(stg, out_hbm.at[cid, pl.ds(base + r * RCH, RCH)])


_sc_segsum = pl.kernel(
    _segsum_body,
    out_type=jax.ShapeDtypeStruct((NC, NP, D), _f32),
    mesh=plsc.VectorSubcoreMesh(core_axis_name="c", subcore_axis_name="s",
                                num_cores=NC, num_subcores=NS),
    scratch_types=(
        [pltpu.VMEM((2, C), jnp.int32)] * 4
        + [pltpu.VMEM((C, D), _f32)] * 3
        + [pltpu.VMEM_SHARED((NP, D), _f32)]
        + [pltpu.SemaphoreType.DMA] * 10
    ),
)


def _count_body(idx_hbm, ones_hbm, zrow_hbm, out_hbm,
                i0, i1, i2, i3, ob, accd,
                is0, is1, is2, is3, ss0, ss1, ss2, ss3):
    cid = lax.axis_index("c")
    sid = lax.axis_index("s")
    wid = sid * NC + cid
    ibufs = (i0, i1, i2, i3)
    isems = (is0, is1, is2, is3)
    ssems = (ss0, ss1, ss2, ss3)

    def idx_load(k):
        return pltpu.async_copy(idx_hbm.at[wid, k], ibufs[k % 4],
                                isems[k % 4])

    def scat_desc(k):
        return pltpu.make_async_copy(ob, accd.at[ibufs[k % 4].at[1]],
                                     ssems[k % 4])

    idx_load(0)
    idx_load(1)
    idx_load(2)
    stg = ob.at[pl.ds(0, RCH)]
    pltpu.sync_copy(zrow_hbm, stg)
    base = sid * RPS
    for r in range(NRCH):
        pltpu.sync_copy(stg, accd.at[pl.ds(base + r * RCH, RCH)])
    pltpu.sync_copy(ones_hbm, ob)
    plsc.subcore_barrier()
    for j in range(NCHUNK):
        pltpu.make_async_copy(idx_hbm.at[wid, j], ibufs[j % 4],
                              isems[j % 4]).wait()
        pltpu.async_copy(ob, accd.at[ibufs[j % 4].at[1]], ssems[j % 4],
                         add=True)
        if j + 3 < NCHUNK:
            if j >= 1:
                scat_desc(j - 1).wait()
            idx_load(j + 3)
    for k in range(max(0, NCHUNK - 4), NCHUNK):
        scat_desc(k).wait()
    plsc.subcore_barrier()
    for r in range(NRCH):
        pltpu.sync_copy(accd.at[pl.ds(base + r * RCH, RCH)], stg)
        pltpu.sync_copy(stg, out_hbm.at[cid, pl.ds(base + r * RCH, RCH)])


_sc_count = pl.kernel(
    _count_body,
    out_type=jax.ShapeDtypeStruct((NC, NP, D), _f32),
    mesh=plsc.VectorSubcoreMesh(core_axis_name="c", subcore_axis_name="s",
                                num_cores=NC, num_subcores=NS),
    scratch_types=(
        [pltpu.VMEM((2, C), jnp.int32)] * 4
        + [pltpu.VMEM((C, D), _f32)]
        + [pltpu.VMEM_SHARED((NP, D), _f32)]
        + [pltpu.SemaphoreType.DMA] * 8
    ),
)


def _dinv_body(deg_ref, hw_ref, dinv_ref, hs_ref):
    dinv = lax.rsqrt(deg_ref[0] + deg_ref[1] + 1.0)
    dinv_ref[...] = dinv
    hs_ref[...] = hw_ref[...] * dinv


_pc_dinv = pl.pallas_call(
    _dinv_body,
    grid=(NBLK,),
    in_specs=[
        pl.BlockSpec((NC, BLK, D), lambda i: (0, i, 0)),
        pl.BlockSpec((BLK, D), lambda i: (i, 0)),
    ],
    out_specs=[
        pl.BlockSpec((BLK, D), lambda i: (i, 0)),
        pl.BlockSpec((BLK, D), lambda i: (i, 0)),
    ],
    out_shape=[
        jax.ShapeDtypeStruct((N, D), _f32),
        jax.ShapeDtypeStruct((N, D), _f32),
    ],
)


def _init_body(x_ref, w0_ref, b0_ref, w1_ref, o_ref):
    h0 = jnp.dot(x_ref[...], w0_ref[...], precision=_PREC,
                 preferred_element_type=_f32) + b0_ref[0:1, :]
    o_ref[...] = jnp.dot(h0, w1_ref[...], precision=_PREC,
                         preferred_element_type=_f32)


_pc_init = pl.pallas_call(
    _init_body,
    grid=(NBLK,),
    in_specs=[
        pl.BlockSpec((BLK, D), lambda i: (i, 0)),
        pl.BlockSpec((D, D), lambda i: (0, 0)),
        pl.BlockSpec((8, D), lambda i: (0, 0)),
        pl.BlockSpec((D, D), lambda i: (0, 0)),
    ],
    out_specs=pl.BlockSpec((BLK, D), lambda i: (i, 0)),
    out_shape=jax.ShapeDtypeStruct((N, D), _f32),
)


def _post_agg(p_ref, hs_ref, dinv_ref, pars_ref):
    agg = dinv_ref[...] * (p_ref[0] + p_ref[1] + hs_ref[...]) + pars_ref[0:1, :]
    mu = jnp.mean(agg, axis=-1, keepdims=True)
    xc = agg - mu
    var = jnp.mean(xc * xc, axis=-1, keepdims=True)
    hn = xc * lax.rsqrt(var + 1e-5) * pars_ref[1:2, :] + pars_ref[2:3, :]
    return jnp.maximum(hn, 0.0)


def _layer_body(p_ref, hs_ref, dinv_ref, w_ref, pars_ref, o_ref):
    h = _post_agg(p_ref, hs_ref, dinv_ref, pars_ref)
    o_ref[...] = jnp.dot(h, w_ref[...], precision=_PREC,
                         preferred_element_type=_f32) * dinv_ref[...]


_pc_layer = pl.pallas_call(
    _layer_body,
    grid=(NBLK,),
    in_specs=[
        pl.BlockSpec((NC, BLK, D), lambda i: (0, i, 0)),
        pl.BlockSpec((BLK, D), lambda i: (i, 0)),
        pl.BlockSpec((BLK, D), lambda i: (i, 0)),
        pl.BlockSpec((D, D), lambda i: (0, 0)),
        pl.BlockSpec((8, D), lambda i: (0, 0)),
    ],
    out_specs=pl.BlockSpec((BLK, D), lambda i: (i, 0)),
    out_shape=jax.ShapeDtypeStruct((N, D), _f32),
)


def _final_body(p_ref, hs_ref, dinv_ref, pars_ref, batch_ref, wf_ref, bf_ref,
                o_ref, pool_ref, cnt_ref):
    i = pl.program_id(0)

    @pl.when(i == 0)
    def _():
        pool_ref[...] = jnp.zeros_like(pool_ref)
        cnt_ref[...] = jnp.zeros_like(cnt_ref)

    h = _post_agg(p_ref, hs_ref, dinv_ref, pars_ref)
    ids = batch_ref[:, 0:1]
    gi = lax.broadcasted_iota(jnp.int32, (BLK, G), 1)
    oh = (ids == gi).astype(_f32)
    dn = (((0,), (0,)), ((), ()))
    pool_ref[...] += lax.dot_general(oh, h, dn, precision=_PREC,
                                     preferred_element_type=_f32)
    cnt_ref[...] += lax.dot_general(oh, jnp.ones((BLK, D), _f32), dn,
                                    precision=_PREC,
                                    preferred_element_type=_f32)

    @pl.when(i == NBLK - 1)
    def _():
        pooled = pool_ref[...] / jnp.maximum(cnt_ref[...], 1.0)
        o_ref[...] = jnp.dot(pooled, wf_ref[...], precision=_PREC,
                             preferred_element_type=_f32) + bf_ref[0:1, :]


_pc_final = pl.pallas_call(
    _final_body,
    grid=(NBLK,),
    in_specs=[
        pl.BlockSpec((NC, BLK, D), lambda i: (0, i, 0)),
        pl.BlockSpec((BLK, D), lambda i: (i, 0)),
        pl.BlockSpec((BLK, D), lambda i: (i, 0)),
        pl.BlockSpec((8, D), lambda i: (0, 0)),
        pl.BlockSpec((BLK, D), lambda i: (i, 0)),
        pl.BlockSpec((D, D), lambda i: (0, 0)),
        pl.BlockSpec((8, D), lambda i: (0, 0)),
    ],
    out_specs=pl.BlockSpec((G, D), lambda i: (0, 0)),
    out_shape=jax.ShapeDtypeStruct((G, D), _f32),
    scratch_shapes=[pltpu.VMEM((G, D), _f32), pltpu.VMEM((G, D), _f32)],
)


def _row8(v):
    return jnp.broadcast_to(v[None, :], (8, D)).astype(_f32)


def _pack3(b, g, be):
    return jnp.concatenate(
        [b[None, :], g[None, :], be[None, :], jnp.zeros((5, D), _f32)], axis=0)


def kernel(x, edge_index, batch, W0, b0, Ws, bs, gammas, betas, Wf, bf):
    srcw = edge_index[0].reshape(NW, EPW)
    dstw = edge_index[1].reshape(NW, EPW)
    wi = jnp.arange(NW, dtype=jnp.int32)[:, None]
    pi = jnp.arange(PAD, dtype=jnp.int32)[None, :]
    src_pad = (wi * 131 + pi * 97) % N
    dst_pad = N + (wi * 7 + pi * 13) % (NP - N)
    srcp = jnp.concatenate([srcw, src_pad], axis=1).reshape(NW, NCHUNK, C)
    dstp = jnp.concatenate([dstw, dst_pad], axis=1).reshape(NW, NCHUNK, C)
    idxc = jnp.stack([srcp, dstp], axis=2)
    zrow = jnp.zeros((RCH, D), _f32)
    batchb = jnp.broadcast_to(batch[:, None], (N, D))

    hw = _pc_init(x, W0, _row8(b0), Ws[0])
    deg2 = _sc_count(idxc, jnp.ones((C, D), _f32), zrow)
    dinv, hs = _pc_dinv(deg2, hw)
    for i in range(NUM_LAYERS):
        p2 = _sc_segsum(hs, idxc, zrow)
        if i + 1 < NUM_LAYERS:
            hs = _pc_layer(p2, hs, dinv, Ws[i + 1],
                           _pack3(bs[i], gammas[i], betas[i]))
        else:
            out = _pc_final(p2, hs, dinv, _pack3(bs[i], gammas[i], betas[i]),
                            batchb, Wf, _row8(bf))
    return out
